# baseline (speedup 1.0000x reference)
import functools

import jax
import jax.numpy as jnp
from jax.experimental import pallas as pl
from jax.experimental.pallas import tpu as pltpu

_NEG_INF = float("-inf")


def _attn_body(mask_ref, hb_ref, hall_ref, aw_ref, out_ref):
    hb = hb_ref[...]
    hall = hall_ref[...]
    aw = aw_ref[...]
    q = hb * aw
    s = jax.lax.dot_general(
        q, hall, (((1,), (1,)), ((), ())),
        preferred_element_type=jnp.float32)
    s = jnp.where(s > 0, s, 0.2 * s)
    mask = mask_ref[...] > 0
    sm = jnp.where(mask, s, _NEG_INF)
    m = jnp.max(sm, axis=1, keepdims=True)
    m = jnp.where(jnp.isfinite(m), m, 0.0)
    e = jnp.where(mask, jnp.exp(s - m), 0.0)
    den = jnp.sum(e, axis=1, keepdims=True)
    alpha = e / jnp.where(den > 0, den, 1.0)
    out_ref[...] = jax.lax.dot_general(
        alpha, hall, (((1,), (0,)), ((), ())),
        preferred_element_type=jnp.float32)


@functools.partial(jax.jit, static_argnames=("interpret",))
def _attention(mask, hidden, a_row, interpret=False):
    n, d = hidden.shape
    br = 256
    grid = (n // br,)
    return pl.pallas_call(
        _attn_body,
        grid=grid,
        in_specs=[
            pl.BlockSpec((br, n), lambda i: (i, 0)),
            pl.BlockSpec((br, d), lambda i: (i, 0)),
            pl.BlockSpec((n, d), lambda i: (0, 0)),
            pl.BlockSpec((1, d), lambda i: (0, 0)),
        ],
        out_specs=pl.BlockSpec((br, d), lambda i: (i, 0)),
        out_shape=jax.ShapeDtypeStruct((n, d), jnp.float32),
        interpret=interpret,
    )(mask, hidden, hidden, a_row)


def kernel(hidden, edge_index, batch, a_w):
    n, d = hidden.shape
    mask = (
        jnp.zeros((n, n), jnp.float32)
        .at[edge_index[0], edge_index[1]].set(1.0)
        .at[edge_index[1], edge_index[0]].set(1.0)
    )
    a_row = a_w.reshape(1, d)
    return _attention(mask, hidden, a_row)


# speedup vs baseline: 2.9917x; 2.9917x over previous
import functools

import jax
import jax.numpy as jnp
from jax import lax
from jax.experimental import pallas as pl
from jax.experimental.pallas import tpu as pltpu
from jax.experimental.pallas import tpu_sc as plsc

_NEG_INF = float("-inf")

N = 4096
D = 64
E = 131072

_NC = 2
_NS = 16
_NW = _NC * _NS
_EPT = E // _NW
_IDX_ROWS = 2 * _EPT // 128


def _scatter_body(edges_hbm, mask_ref, src_v, dst_v, idx_v, ones_v, sem):
    c = lax.axis_index("c")
    s = lax.axis_index("s")
    wid = s * _NC + c
    base = wid * _EPT
    pltpu.sync_copy(edges_hbm.at[pl.ds(base, _EPT)], src_v)
    pltpu.sync_copy(edges_hbm.at[pl.ds(E + base, _EPT)], dst_v)

    @pl.loop(0, 8)
    def _fill_ones(k):
        ones_v[pl.ds(k * 16, 16)] = jnp.ones((16,), jnp.float32)

    @pl.loop(0, _EPT // 16)
    def _build(i):
        sv = src_v[pl.ds(i * 16, 16)]
        dv = dst_v[pl.ds(i * 16, 16)]
        f1 = sv * N + dv
        f2 = dv * N + sv
        row = i // 4
        col = (i % 4) * 32
        idx_v[row, pl.ds(col, 16)] = f1
        idx_v[row, pl.ds(col + 16, 16)] = f2

    @pl.loop(0, _IDX_ROWS // 16)
    def _scatter(j):
        copies = []
        for k in range(16):
            copies.append(pltpu.make_async_copy(
                ones_v, mask_ref.at[idx_v.at[j * 16 + k]], sem))
        for cpy in copies:
            cpy.start()
        for cpy in copies:
            cpy.wait()


def _build_mask(edges_flat, mask_ref):
    mesh = plsc.VectorSubcoreMesh(core_axis_name="c", subcore_axis_name="s")
    f = pl.kernel(
        _scatter_body,
        out_type=(),
        mesh=mesh,
        scratch_types=[
            pltpu.VMEM((_EPT,), jnp.int32),
            pltpu.VMEM((_EPT,), jnp.int32),
            pltpu.VMEM((_IDX_ROWS, 128), jnp.int32),
            pltpu.VMEM((128,), jnp.float32),
            pltpu.SemaphoreType.DMA,
        ],
    )
    f(edges_flat, mask_ref)


def _attn_body(mask_ref, hb_ref, hall_ref, aw_ref, out_ref):
    hb = hb_ref[...]
    hall = hall_ref[...]
    aw = aw_ref[...]
    q = hb * aw
    s = jax.lax.dot_general(
        q, hall, (((1,), (1,)), ((), ())),
        preferred_element_type=jnp.float32)
    s = jnp.where(s > 0, s, 0.2 * s)
    mask = mask_ref[...] > 0
    sm = jnp.where(mask, s, _NEG_INF)
    m = jnp.max(sm, axis=1, keepdims=True)
    m = jnp.where(jnp.isfinite(m), m, 0.0)
    e = jnp.where(mask, jnp.exp(s - m), 0.0)
    den = jnp.sum(e, axis=1, keepdims=True)
    alpha = e / jnp.where(den > 0, den, 1.0)
    out_ref[...] = jax.lax.dot_general(
        alpha, hall, (((1,), (0,)), ((), ())),
        preferred_element_type=jnp.float32)


def _attention(mask, hidden, a_row):
    n, d = hidden.shape
    br = 256
    grid = (n // br,)
    return pl.pallas_call(
        _attn_body,
        grid=grid,
        in_specs=[
            pl.BlockSpec((br, n), lambda i: (i, 0)),
            pl.BlockSpec((br, d), lambda i: (i, 0)),
            pl.BlockSpec((n, d), lambda i: (0, 0)),
            pl.BlockSpec((1, d), lambda i: (0, 0)),
        ],
        out_specs=pl.BlockSpec((br, d), lambda i: (i, 0)),
        out_shape=jax.ShapeDtypeStruct((n, d), jnp.float32),
    )(mask, hidden, hidden, a_row)


@jax.jit
def _run(hidden, edge_index, a_w):
    n, d = hidden.shape
    edges_flat = edge_index.reshape(-1).astype(jnp.int32)
    mask_ref = jax.new_ref(jnp.zeros((n * n,), jnp.float32))
    _build_mask(edges_flat, mask_ref)
    mask = mask_ref[...].reshape(n, n)
    a_row = a_w.reshape(1, d)
    return _attention(mask, hidden, a_row)


def kernel(hidden, edge_index, batch, a_w):
    return _run(hidden, edge_index, a_w)
